# baseline (speedup 1.0000x reference)
import functools

import jax
import jax.numpy as jnp
from jax import lax
from jax.experimental import pallas as pl
from jax.experimental.pallas import tpu as pltpu
from jax.experimental.pallas import tpu_sc as plsc

NN = 10000
NP = 10240
EE = 320000
NC = 2
NS = 16
LANES = 16
BLK = 128
NBLK = EE // BLK
ROWS_PER_TEC = NP // NS
ZROWS = 128
TCBLK = 1000


def _make_att_kernel():
    per_sc = NBLK // NC
    mesh = plsc.VectorSubcoreMesh(core_axis_name="c", subcore_axis_name="s")

    @functools.partial(
        pl.kernel,
        out_type=(
            jax.ShapeDtypeStruct((EE, 16), jnp.float32),
            jax.ShapeDtypeStruct((NC, NP, 16), jnp.float32),
        ),
        mesh=mesh,
        scratch_types=(
            pltpu.VMEM((BLK,), jnp.int32),
            pltpu.VMEM((BLK,), jnp.int32),
            pltpu.VMEM((BLK, 16), jnp.float32),
            pltpu.VMEM((BLK, 16), jnp.float32),
            pltpu.VMEM((BLK, 16), jnp.float32),
            pltpu.VMEM((ZROWS, 16), jnp.float32),
            pltpu.VMEM_SHARED((NP, 16), jnp.float32),
            pltpu.SemaphoreType.DMA,
        ),
        compiler_params=pltpu.CompilerParams(use_tc_tiling_on_sc=False, needs_layout_passes=False),
    )
    def att(als_hbm, ald_hbm, src_hbm, dst_hbm, w_hbm, den_hbm,
            src_v, dst_v, asrc_v, adst_v, wrow_v, zero_v, den_sh, sem):
        core = lax.axis_index("c")
        sub = lax.axis_index("s")

        def zinit(i, _):
            zero_v[i, :] = jnp.zeros((LANES,), jnp.float32)
            return 0
        lax.fori_loop(0, ZROWS, zinit, 0)
        row0 = sub * ROWS_PER_TEC
        for k in range(ROWS_PER_TEC // ZROWS):
            pltpu.sync_copy(zero_v, den_sh.at[pl.ds(row0 + k * ZROWS, ZROWS)])
        plsc.subcore_barrier()

        blo = core * per_sc + (sub * per_sc) // NS
        bhi = core * per_sc + ((sub + 1) * per_sc) // NS

        def blk_loop(b, _):
            base = b * BLK
            pltpu.sync_copy(src_hbm.at[pl.ds(base, BLK)], src_v)
            pltpu.sync_copy(dst_hbm.at[pl.ds(base, BLK)], dst_v)
            pltpu.async_copy(als_hbm.at[src_v], asrc_v, sem).wait()
            pltpu.async_copy(ald_hbm.at[dst_v], adst_v, sem).wait()

            def edge(i, _):
                s = asrc_v[i, :] + adst_v[i, :]
                wrow_v[i, :] = jnp.exp(jnp.maximum(s, 0.2 * s))
                return 0
            lax.fori_loop(0, BLK, edge, 0, unroll=8)
            pltpu.sync_copy(wrow_v, w_hbm.at[pl.ds(base, BLK)])
            pltpu.sync_copy(wrow_v, den_sh.at[dst_v], add=True)
            return 0
        lax.fori_loop(blo, bhi, blk_loop, 0)
        plsc.subcore_barrier()
        for k in range(ROWS_PER_TEC // ZROWS):
            r = row0 + k * ZROWS
            pltpu.sync_copy(den_sh.at[pl.ds(r, ZROWS)],
                            den_hbm.at[core, pl.ds(r, ZROWS)])

    return att


def _make_msg_kernel(nslab, cols, slabs_per_core, heads_per_slab, edge_split):
    vpr = cols // LANES
    vph = vpr // heads_per_slab
    n_out = NC if edge_split else nslab
    per_loop = NBLK // NC if edge_split else NBLK
    mesh = plsc.VectorSubcoreMesh(core_axis_name="c", subcore_axis_name="s")

    @functools.partial(
        pl.kernel,
        out_type=jax.ShapeDtypeStruct((n_out, NP, cols), jnp.float32),
        mesh=mesh,
        scratch_types=(
            pltpu.VMEM((BLK,), jnp.int32),
            pltpu.VMEM((BLK,), jnp.int32),
            pltpu.VMEM((BLK,), jnp.int32),
            pltpu.VMEM((BLK, 16), jnp.float32),
            pltpu.VMEM((BLK, cols), jnp.float32),
            pltpu.VMEM((ZROWS, cols), jnp.float32),
            pltpu.VMEM_SHARED((NP, cols), jnp.float32),
            pltpu.SemaphoreType.DMA,
        ),
        compiler_params=pltpu.CompilerParams(use_tc_tiling_on_sc=False, needs_layout_passes=False),
    )
    def msg(h_hbm, src_hbm, dst_hbm, w_hbm, out_hbm,
            src_v, dst_v, gidx_v, wrow_v, rows_v, zero_v, acc_sh, sem):
        core = lax.axis_index("c")
        sub = lax.axis_index("s")
        row0 = sub * ROWS_PER_TEC

        def zinit(i, _):
            for j in range(vpr):
                zero_v[i, pl.ds(j * LANES, LANES)] = jnp.zeros((LANES,),
                                                               jnp.float32)
            return 0
        lax.fori_loop(0, ZROWS, zinit, 0)

        for si in range(slabs_per_core):
            slab = 0 if edge_split else core * slabs_per_core + si
            for k in range(ROWS_PER_TEC // ZROWS):
                pltpu.sync_copy(zero_v,
                                acc_sh.at[pl.ds(row0 + k * ZROWS, ZROWS)])
            plsc.subcore_barrier()

            boff = core * per_loop if edge_split else 0
            blo = boff + (sub * per_loop) // NS
            bhi = boff + ((sub + 1) * per_loop) // NS
            h0 = slab * heads_per_slab

            def blk_loop(b, _):
                base = b * BLK
                pltpu.sync_copy(src_hbm.at[pl.ds(base, BLK)], src_v)
                pltpu.sync_copy(dst_hbm.at[pl.ds(base, BLK)], dst_v)
                if nslab == 1:
                    idx_ref = src_v
                else:
                    for k in range(BLK // LANES):
                        sl = pl.ds(k * LANES, LANES)
                        gidx_v[sl] = src_v[sl] * nslab + slab
                    idx_ref = gidx_v
                pltpu.async_copy(h_hbm.at[idx_ref], rows_v, sem).wait()
                pltpu.sync_copy(w_hbm.at[pl.ds(base, BLK)], wrow_v)

                def edge(i, _):
                    for hh in range(heads_per_slab):
                        wb = plsc.load_gather(
                            wrow_v,
                            [jnp.full((LANES,), i, jnp.int32),
                             jnp.full((LANES,), h0 + hh, jnp.int32)])
                        for j in range(vph):
                            sl = pl.ds((hh * vph + j) * LANES, LANES)
                            rows_v[i, sl] = rows_v[i, sl] * wb
                    return 0
                lax.fori_loop(0, BLK, edge, 0, unroll=4)
                pltpu.sync_copy(rows_v, acc_sh.at[dst_v], add=True)
                return 0
            lax.fori_loop(blo, bhi, blk_loop, 0)
            plsc.subcore_barrier()

            out_maj = core if edge_split else slab
            for k in range(ROWS_PER_TEC // ZROWS):
                r = row0 + k * ZROWS
                pltpu.sync_copy(acc_sh.at[pl.ds(r, ZROWS)],
                                out_hbm.at[out_maj, pl.ds(r, ZROWS)])

    return msg


def _tc_first(x, W, As2, Ad2):
    hd = W.shape[1]

    def body(x_ref, w_ref, as_ref, ad_ref, h_ref, als_ref, ald_ref):
        h = jnp.dot(x_ref[...], w_ref[...], preferred_element_type=jnp.float32)
        h_ref[...] = h
        als_ref[...] = jnp.dot(h, as_ref[...],
                               preferred_element_type=jnp.float32)
        ald_ref[...] = jnp.dot(h, ad_ref[...],
                               preferred_element_type=jnp.float32)

    return pl.pallas_call(
        body,
        grid=(NN // TCBLK,),
        in_specs=[
            pl.BlockSpec((TCBLK, x.shape[1]), lambda i: (i, 0)),
            pl.BlockSpec(W.shape, lambda i: (0, 0)),
            pl.BlockSpec(As2.shape, lambda i: (0, 0)),
            pl.BlockSpec(Ad2.shape, lambda i: (0, 0)),
        ],
        out_specs=[
            pl.BlockSpec((TCBLK, hd), lambda i: (i, 0)),
            pl.BlockSpec((TCBLK, 16), lambda i: (i, 0)),
            pl.BlockSpec((TCBLK, 16), lambda i: (i, 0)),
        ],
        out_shape=[
            jax.ShapeDtypeStruct((NN, hd), jnp.float32),
            jax.ShapeDtypeStruct((NN, 16), jnp.float32),
            jax.ShapeDtypeStruct((NN, 16), jnp.float32),
        ],
    )(x, W, As2, Ad2)


def _tc_mid(acc, den, exp8, bvec, W, As2, Ad2, sum_parts):
    P = acc.shape[0]
    C = acc.shape[2]
    hd_prev = exp8.shape[1]
    hd = W.shape[1]

    def body(acc_ref, den_ref, e8_ref, b_ref, w_ref, as_ref, ad_ref,
             h_ref, als_ref, ald_ref):
        a = acc_ref[...]
        if sum_parts:
            g = a[0] + a[1]
        else:
            g = jnp.concatenate([a[p] for p in range(P)], axis=-1)
        d = den_ref[...]
        den8 = d[0, :, 0:8] + d[1, :, 0:8]
        denr = jnp.dot(den8, e8_ref[...], preferred_element_type=jnp.float32)
        g = g / (denr + 1e-16) + b_ref[...]
        g = jnp.where(g > 0, g, jnp.exp(g) - 1.0)
        h = jnp.dot(g, w_ref[...], preferred_element_type=jnp.float32)
        h_ref[...] = h
        als_ref[...] = jnp.dot(h, as_ref[...],
                               preferred_element_type=jnp.float32)
        ald_ref[...] = jnp.dot(h, ad_ref[...],
                               preferred_element_type=jnp.float32)

    return pl.pallas_call(
        body,
        grid=(NN // TCBLK,),
        in_specs=[
            pl.BlockSpec((P, TCBLK, C), lambda i: (0, i, 0)),
            pl.BlockSpec((NC, TCBLK, 16), lambda i: (0, i, 0)),
            pl.BlockSpec(exp8.shape, lambda i: (0, 0)),
            pl.BlockSpec((1, hd_prev), lambda i: (0, 0)),
            pl.BlockSpec(W.shape, lambda i: (0, 0)),
            pl.BlockSpec(As2.shape, lambda i: (0, 0)),
            pl.BlockSpec(Ad2.shape, lambda i: (0, 0)),
        ],
        out_specs=[
            pl.BlockSpec((TCBLK, hd), lambda i: (i, 0)),
            pl.BlockSpec((TCBLK, 16), lambda i: (i, 0)),
            pl.BlockSpec((TCBLK, 16), lambda i: (i, 0)),
        ],
        out_shape=[
            jax.ShapeDtypeStruct((NN, hd), jnp.float32),
            jax.ShapeDtypeStruct((NN, 16), jnp.float32),
            jax.ShapeDtypeStruct((NN, 16), jnp.float32),
        ],
    )(acc, den, exp8, bvec, W, As2, Ad2)


def _tc_final(acc4, den4, b4p, batch3, Wih_p, bl, W_fc, bfc):
    nb = NN // TCBLK

    def body(acc_ref, den_ref, b4_ref, bat_ref, wih_ref, bl_ref, wfc_ref,
             bfc_ref, out_ref, sums_ref, cnts_ref):
        i = pl.program_id(0)

        @pl.when(i == 0)
        def _():
            sums_ref[...] = jnp.zeros_like(sums_ref)
            cnts_ref[...] = jnp.zeros_like(cnts_ref)

        a = acc_ref[...]
        d = den_ref[...]
        den = d[0, :, 0:1] + d[1, :, 0:1]
        g = a[0] / (den + 1e-16) + b4_ref[...]
        g = jnp.where(g > 0, g, jnp.exp(g) - 1.0)
        b = bat_ref[0, 0, :]
        oh = (b[:, None] == lax.broadcasted_iota(jnp.int32, (TCBLK, 16), 1)
              ).astype(jnp.float32)
        dn = (((0,), (0,)), ((), ()))
        sums_ref[...] += lax.dot_general(oh, g, dn,
                                         preferred_element_type=jnp.float32)
        cnts_ref[...] += lax.dot_general(oh, jnp.ones_like(g), dn,
                                         preferred_element_type=jnp.float32)

        @pl.when(i == nb - 1)
        def _():
            emb = sums_ref[...] / jnp.maximum(cnts_ref[...], 1.0)
            dt = (((1,), (1,)), ((), ()))
            gates = lax.dot_general(emb, wih_ref[...], dt,
                                    preferred_element_type=jnp.float32)
            gates = gates + bl_ref[...]
            i_g = gates[:, 0:128]
            g_g = gates[:, 256:384]
            o_g = gates[:, 384:512]
            c = jax.nn.sigmoid(i_g) * jnp.tanh(g_g)
            hh = jax.nn.sigmoid(o_g) * jnp.tanh(c)
            out_ref[...] = lax.dot_general(hh, wfc_ref[...], dt,
                                           preferred_element_type=jnp.float32
                                           ) + bfc_ref[...]

    def accsum(acc_ref, out_ref):
        a = acc_ref[...]
        out_ref[...] = (a[0] + a[1])[None]

    acc_s = pl.pallas_call(
        accsum,
        grid=(nb,),
        in_specs=[pl.BlockSpec((NC, TCBLK, 16), lambda i: (0, i, 0))],
        out_specs=pl.BlockSpec((1, TCBLK, 16), lambda i: (0, i, 0)),
        out_shape=jax.ShapeDtypeStruct((1, NN, 16), jnp.float32),
    )(acc4)

    return pl.pallas_call(
        body,
        grid=(nb,),
        in_specs=[
            pl.BlockSpec((1, TCBLK, 16), lambda i: (0, i, 0)),
            pl.BlockSpec((NC, TCBLK, 16), lambda i: (0, i, 0)),
            pl.BlockSpec((1, 16), lambda i: (0, 0)),
            pl.BlockSpec((1, 1, TCBLK), lambda i: (i, 0, 0)),
            pl.BlockSpec(Wih_p.shape, lambda i: (0, 0)),
            pl.BlockSpec((1, 512), lambda i: (0, 0)),
            pl.BlockSpec(W_fc.shape, lambda i: (0, 0)),
            pl.BlockSpec((1, 16), lambda i: (0, 0)),
        ],
        out_specs=pl.BlockSpec((16, 16), lambda i: (0, 0)),
        out_shape=jax.ShapeDtypeStruct((16, 16), jnp.float32),
        scratch_shapes=[
            pltpu.VMEM((16, 16), jnp.float32),
            pltpu.VMEM((16, 16), jnp.float32),
        ],
    )(acc_s, den4, b4p, batch3, Wih_p, bl, W_fc, bfc)


def _mk_as2(a, hd):
    heads, dout = a.shape
    eye = jnp.eye(8, dtype=jnp.float32)[:heads]
    blockdiag = (a[:, :, None] * eye[:, None, :]).reshape(heads * dout, 8)
    if heads * dout < hd:
        blockdiag = jnp.pad(blockdiag, ((0, hd - heads * dout), (0, 0)))
    return jnp.concatenate([blockdiag, blockdiag], axis=1)


def _mk_exp8(heads, dout, hd):
    col_head = jnp.minimum(jnp.arange(hd) // dout, heads - 1)
    return (jnp.arange(8)[:, None] == col_head[None, :]).astype(jnp.float32)


def kernel(x, edge_index, batch, W1, as1, ad1, b1, W2, as2, ad2, b2, W3, as3,
           ad3, b3, W4, as4, ad4, b4, W_ih, W_hh, b_ih, b_hh, W_fc, b_fc):
    f32 = jnp.float32
    src = edge_index[0].astype(jnp.int32)
    dst = edge_index[1].astype(jnp.int32)
    batch3 = batch.astype(jnp.int32).reshape(NN // TCBLK, 1, TCBLK)

    W4p = jnp.pad(W4, ((0, 0), (0, 8)))
    b4p = jnp.pad(b4, (0, 8)).reshape(1, 16).astype(f32)
    Wih_p = jnp.pad(W_ih, ((0, 0), (0, 8))).astype(f32)
    bl = (b_ih + b_hh).reshape(1, 512).astype(f32)
    bfc = b_fc.reshape(1, 16).astype(f32)

    att = _make_att_kernel()
    msg1 = _make_msg_kernel(4, 128, 2, 2, False)
    msg2 = _make_msg_kernel(2, 128, 1, 4, False)
    msg3 = _make_msg_kernel(1, 128, 1, 8, True)
    msg4 = _make_msg_kernel(1, 16, 1, 1, True)

    h1, als1, ald1 = _tc_first(x, W1, _mk_as2(as1, 512), _mk_as2(ad1, 512))
    w1, den1 = att(als1, ald1, src, dst)
    acc1 = msg1(h1.reshape(NN * 4, 128), src, dst, w1)

    h2, als2, ald2 = _tc_mid(acc1, den1, _mk_exp8(8, 64, 512),
                             b1.reshape(1, 512), W2,
                             _mk_as2(as2, 256), _mk_as2(ad2, 256), False)
    w2, den2 = att(als2, ald2, src, dst)
    acc2 = msg2(h2.reshape(NN * 2, 128), src, dst, w2)

    h3, als3, ald3 = _tc_mid(acc2, den2, _mk_exp8(8, 32, 256),
                             b2.reshape(1, 256), W3,
                             _mk_as2(as3, 128), _mk_as2(ad3, 128), False)
    w3, den3 = att(als3, ald3, src, dst)
    acc3 = msg3(h3, src, dst, w3)

    h4, als4, ald4 = _tc_mid(acc3, den3, _mk_exp8(8, 16, 128),
                             b3.reshape(1, 128), W4p,
                             _mk_as2(as4, 16), _mk_as2(ad4, 16), True)
    w4, den4 = att(als4, ald4, src, dst)
    acc4 = msg4(h4, src, dst, w4)

    return _tc_final(acc4, den4, b4p, batch3, Wih_p, bl, W_fc, bfc)


# speedup vs baseline: 1.3205x; 1.3205x over previous
import functools

import jax
import jax.numpy as jnp
from jax import lax
from jax.experimental import pallas as pl
from jax.experimental.pallas import tpu as pltpu
from jax.experimental.pallas import tpu_sc as plsc

NN = 10000
NP = 10240
EE = 320000
NC = 2
NS = 16
LANES = 16
BLK = 128
EPB = 2560
EP = EPB * BLK
ROWS_PER_TEC = NP // NS
ZROWS = 128
TCBLK = 1000

_SC_PARAMS = pltpu.CompilerParams(use_tc_tiling_on_sc=False,
                                  needs_layout_passes=False)


def _make_att_kernel():
    nb = EPB // (NC * NS)
    mesh = plsc.VectorSubcoreMesh(core_axis_name="c", subcore_axis_name="s")

    @functools.partial(
        pl.kernel,
        out_type=(
            jax.ShapeDtypeStruct((EP, 16), jnp.float32),
            jax.ShapeDtypeStruct((NC, NP, 16), jnp.float32),
        ),
        mesh=mesh,
        scratch_types=(
            pltpu.VMEM((2, BLK), jnp.int32),
            pltpu.VMEM((2, BLK), jnp.int32),
            pltpu.VMEM((BLK, 16), jnp.float32),
            pltpu.VMEM((BLK, 16), jnp.float32),
            pltpu.VMEM((BLK, 16), jnp.float32),
            pltpu.VMEM((BLK, 16), jnp.float32),
            pltpu.VMEM((BLK, 16), jnp.float32),
            pltpu.VMEM((BLK, 16), jnp.float32),
            pltpu.VMEM_SHARED((NP, 16), jnp.float32),
            pltpu.SemaphoreType.DMA,
            pltpu.SemaphoreType.DMA,
            pltpu.SemaphoreType.DMA,
            pltpu.SemaphoreType.DMA,
        ),
        compiler_params=_SC_PARAMS,
    )
    def att(als_hbm, ald_hbm, sd_hbm, w_hbm, den_hbm,
            sd0, sd1, as0, as1, ad0, ad1, wr0, wr1, den_sh,
            ss0, ss1, sa0, sa1):
        core = lax.axis_index("c")
        sub = lax.axis_index("s")

        def zinit(i, _):
            as0[i, :] = jnp.zeros((LANES,), jnp.float32)
            return 0
        lax.fori_loop(0, ZROWS, zinit, 0)
        row0 = sub * ROWS_PER_TEC
        for k in range(ROWS_PER_TEC // ZROWS):
            pltpu.sync_copy(as0, den_sh.at[pl.ds(row0 + k * ZROWS, ZROWS)])
        plsc.subcore_barrier()

        blo = (core * NS + sub) * nb
        sets = ((sd0, as0, ad0, wr0, ss0, sa0),
                (sd1, as1, ad1, wr1, ss1, sa1))

        def issue(b, st):
            sd, asv, adv, _, s_s, s_a = st
            pltpu.sync_copy(sd_hbm.at[b], sd)
            pltpu.async_copy(als_hbm.at[sd.at[0]], asv, s_s)
            pltpu.async_copy(ald_hbm.at[sd.at[1]], adv, s_a)

        def process(b, st):
            sd, asv, adv, wr, s_s, s_a = st
            pltpu.make_async_copy(als_hbm.at[sd.at[0]], asv, s_s).wait()
            pltpu.make_async_copy(ald_hbm.at[sd.at[1]], adv, s_a).wait()

            def edge(i, _):
                s = asv[i, :] + adv[i, :]
                wr[i, :] = jnp.exp(jnp.maximum(s, 0.2 * s))
                return 0
            lax.fori_loop(0, BLK, edge, 0, unroll=4)
            pltpu.sync_copy(wr, w_hbm.at[pl.ds(b * BLK, BLK)])
            pltpu.sync_copy(wr, den_sh.at[sd.at[1]], add=True)

        issue(blo, sets[0])
        issue(blo + 1, sets[1])

        def pair(i, _):
            b0 = blo + 2 * i
            process(b0, sets[0])

            @pl.when(i < nb // 2 - 1)
            def _():
                issue(b0 + 2, sets[0])
            process(b0 + 1, sets[1])

            @pl.when(i < nb // 2 - 1)
            def _():
                issue(b0 + 3, sets[1])
            return 0
        lax.fori_loop(0, nb // 2, pair, 0)
        plsc.subcore_barrier()
        for k in range(ROWS_PER_TEC // ZROWS):
            r = row0 + k * ZROWS
            pltpu.sync_copy(den_sh.at[pl.ds(r, ZROWS)],
                            den_hbm.at[core, pl.ds(r, ZROWS)])

    return att


def _make_msg_kernel(nslab, cols, slabs_per_core, heads_per_slab, edge_split):
    vpr = cols // LANES
    vph = vpr // heads_per_slab
    n_out = NC if edge_split else nslab
    nb = EPB // (NC * NS) if edge_split else EPB // NS
    mesh = plsc.VectorSubcoreMesh(core_axis_name="c", subcore_axis_name="s")

    @functools.partial(
        pl.kernel,
        out_type=jax.ShapeDtypeStruct((n_out, NP, cols), jnp.float32),
        mesh=mesh,
        scratch_types=(
            pltpu.VMEM((2, BLK), jnp.int32),
            pltpu.VMEM((2, BLK), jnp.int32),
            pltpu.VMEM((BLK,), jnp.int32),
            pltpu.VMEM((BLK,), jnp.int32),
            pltpu.VMEM((BLK, 16), jnp.float32),
            pltpu.VMEM((BLK, 16), jnp.float32),
            pltpu.VMEM((BLK, cols), jnp.float32),
            pltpu.VMEM((BLK, cols), jnp.float32),
            pltpu.VMEM_SHARED((NP, cols), jnp.float32),
            pltpu.SemaphoreType.DMA,
            pltpu.SemaphoreType.DMA,
            pltpu.SemaphoreType.DMA,
            pltpu.SemaphoreType.DMA,
        ),
        compiler_params=_SC_PARAMS,
    )
    def msg(h_hbm, sd_hbm, w_hbm, out_hbm,
            sd0, sd1, gx0, gx1, wr0, wr1, rows0, rows1, acc_sh,
            sr0, sr1, sw0, sw1):
        core = lax.axis_index("c")
        sub = lax.axis_index("s")
        row0 = sub * ROWS_PER_TEC

        sets = ((sd0, gx0, wr0, rows0, sr0, sw0),
                (sd1, gx1, wr1, rows1, sr1, sw1))

        for si in range(slabs_per_core):
            slab = 0 if edge_split else core * slabs_per_core + si

            def zinit(i, _):
                for j in range(vpr):
                    rows0[i, pl.ds(j * LANES, LANES)] = jnp.zeros(
                        (LANES,), jnp.float32)
                return 0
            lax.fori_loop(0, ZROWS, zinit, 0)
            for k in range(ROWS_PER_TEC // ZROWS):
                pltpu.sync_copy(rows0,
                                acc_sh.at[pl.ds(row0 + k * ZROWS, ZROWS)])
            plsc.subcore_barrier()

            blo = ((core * NS + sub) * nb if edge_split else sub * nb)
            h0 = slab * heads_per_slab

            def gidx_ref(st):
                sd, gx = st[0], st[1]
                if nslab == 1:
                    return sd.at[0]
                return gx

            def issue(b, st):
                sd, gx, wr, rows, s_r, s_w = st
                pltpu.sync_copy(sd_hbm.at[b], sd)
                if nslab != 1:
                    for k in range(BLK // LANES):
                        sl = pl.ds(k * LANES, LANES)
                        gx[sl] = sd[0, sl] * nslab + slab
                pltpu.async_copy(h_hbm.at[gidx_ref(st)], rows, s_r)
                pltpu.async_copy(w_hbm.at[pl.ds(b * BLK, BLK)], wr, s_w)

            def process(b, st):
                sd, gx, wr, rows, s_r, s_w = st
                pltpu.make_async_copy(h_hbm.at[gidx_ref(st)], rows,
                                      s_r).wait()
                pltpu.make_async_copy(w_hbm.at[pl.ds(b * BLK, BLK)], wr,
                                      s_w).wait()

                def edge(i, _):
                    for hh in range(heads_per_slab):
                        wb = plsc.load_gather(
                            wr,
                            [jnp.full((LANES,), i, jnp.int32),
                             jnp.full((LANES,), h0 + hh, jnp.int32)])
                        for j in range(vph):
                            sl = pl.ds((hh * vph + j) * LANES, LANES)
                            rows[i, sl] = rows[i, sl] * wb
                    return 0
                lax.fori_loop(0, BLK, edge, 0, unroll=2)
                pltpu.sync_copy(rows, acc_sh.at[sd.at[1]], add=True)

            issue(blo, sets[0])
            issue(blo + 1, sets[1])

            def pair(i, _):
                b0 = blo + 2 * i
                process(b0, sets[0])

                @pl.when(i < nb // 2 - 1)
                def _():
                    issue(b0 + 2, sets[0])
                process(b0 + 1, sets[1])

                @pl.when(i < nb // 2 - 1)
                def _():
                    issue(b0 + 3, sets[1])
                return 0
            lax.fori_loop(0, nb // 2, pair, 0)
            plsc.subcore_barrier()

            out_maj = core if edge_split else slab
            for k in range(ROWS_PER_TEC // ZROWS):
                r = row0 + k * ZROWS
                pltpu.sync_copy(acc_sh.at[pl.ds(r, ZROWS)],
                                out_hbm.at[out_maj, pl.ds(r, ZROWS)])

    return msg


def _tc_first(x, W, As2, Ad2):
    hd = W.shape[1]

    def body(x_ref, w_ref, as_ref, ad_ref, h_ref, als_ref, ald_ref):
        h = jnp.dot(x_ref[...], w_ref[...], preferred_element_type=jnp.float32)
        h_ref[...] = h
        als_ref[...] = jnp.dot(h, as_ref[...],
                               preferred_element_type=jnp.float32)
        ald_ref[...] = jnp.dot(h, ad_ref[...],
                               preferred_element_type=jnp.float32)

    return pl.pallas_call(
        body,
        grid=(NN // TCBLK,),
        in_specs=[
            pl.BlockSpec((TCBLK, x.shape[1]), lambda i: (i, 0)),
            pl.BlockSpec(W.shape, lambda i: (0, 0)),
            pl.BlockSpec(As2.shape, lambda i: (0, 0)),
            pl.BlockSpec(Ad2.shape, lambda i: (0, 0)),
        ],
        out_specs=[
            pl.BlockSpec((TCBLK, hd), lambda i: (i, 0)),
            pl.BlockSpec((TCBLK, 16), lambda i: (i, 0)),
            pl.BlockSpec((TCBLK, 16), lambda i: (i, 0)),
        ],
        out_shape=[
            jax.ShapeDtypeStruct((NN, hd), jnp.float32),
            jax.ShapeDtypeStruct((NN, 16), jnp.float32),
            jax.ShapeDtypeStruct((NN, 16), jnp.float32),
        ],
    )(x, W, As2, Ad2)


def _tc_mid(acc, den, exp8, bvec, W, As2, Ad2, sum_parts):
    P = acc.shape[0]
    C = acc.shape[2]
    hd_prev = exp8.shape[1]
    hd = W.shape[1]

    def body(acc_ref, den_ref, e8_ref, b_ref, w_ref, as_ref, ad_ref,
             h_ref, als_ref, ald_ref):
        a = acc_ref[...]
        if sum_parts:
            g = a[0] + a[1]
        else:
            g = jnp.concatenate([a[p] for p in range(P)], axis=-1)
        d = den_ref[...]
        den8 = d[0, :, 0:8] + d[1, :, 0:8]
        denr = jnp.dot(den8, e8_ref[...], preferred_element_type=jnp.float32)
        g = g / (denr + 1e-16) + b_ref[...]
        g = jnp.where(g > 0, g, jnp.exp(g) - 1.0)
        h = jnp.dot(g, w_ref[...], preferred_element_type=jnp.float32)
        h_ref[...] = h
        als_ref[...] = jnp.dot(h, as_ref[...],
                               preferred_element_type=jnp.float32)
        ald_ref[...] = jnp.dot(h, ad_ref[...],
                               preferred_element_type=jnp.float32)

    return pl.pallas_call(
        body,
        grid=(NN // TCBLK,),
        in_specs=[
            pl.BlockSpec((P, TCBLK, C), lambda i: (0, i, 0)),
            pl.BlockSpec((NC, TCBLK, 16), lambda i: (0, i, 0)),
            pl.BlockSpec(exp8.shape, lambda i: (0, 0)),
            pl.BlockSpec((1, hd_prev), lambda i: (0, 0)),
            pl.BlockSpec(W.shape, lambda i: (0, 0)),
            pl.BlockSpec(As2.shape, lambda i: (0, 0)),
            pl.BlockSpec(Ad2.shape, lambda i: (0, 0)),
        ],
        out_specs=[
            pl.BlockSpec((TCBLK, hd), lambda i: (i, 0)),
            pl.BlockSpec((TCBLK, 16), lambda i: (i, 0)),
            pl.BlockSpec((TCBLK, 16), lambda i: (i, 0)),
        ],
        out_shape=[
            jax.ShapeDtypeStruct((NN, hd), jnp.float32),
            jax.ShapeDtypeStruct((NN, 16), jnp.float32),
            jax.ShapeDtypeStruct((NN, 16), jnp.float32),
        ],
    )(acc, den, exp8, bvec, W, As2, Ad2)


def _tc_final(acc4, den4, b4p, batch3, Wih_p, bl, W_fc, bfc):
    nb = NN // TCBLK

    def body(acc_ref, den_ref, b4_ref, bat_ref, wih_ref, bl_ref, wfc_ref,
             bfc_ref, out_ref, sums_ref, cnts_ref):
        i = pl.program_id(0)

        @pl.when(i == 0)
        def _():
            sums_ref[...] = jnp.zeros_like(sums_ref)
            cnts_ref[...] = jnp.zeros_like(cnts_ref)

        d = den_ref[...]
        den = d[0, :, 0:1] + d[1, :, 0:1]
        g = acc_ref[...][0] / (den + 1e-16) + b4_ref[...]
        g = jnp.where(g > 0, g, jnp.exp(g) - 1.0)
        b = bat_ref[0, 0, :]
        oh = (b[:, None] == lax.broadcasted_iota(jnp.int32, (TCBLK, 16), 1)
              ).astype(jnp.float32)
        dn = (((0,), (0,)), ((), ()))
        sums_ref[...] += lax.dot_general(oh, g, dn,
                                         preferred_element_type=jnp.float32)
        cnts_ref[...] += lax.dot_general(oh, jnp.ones_like(g), dn,
                                         preferred_element_type=jnp.float32)

        @pl.when(i == nb - 1)
        def _():
            emb = sums_ref[...] / jnp.maximum(cnts_ref[...], 1.0)
            dt = (((1,), (1,)), ((), ()))
            gates = lax.dot_general(emb, wih_ref[...], dt,
                                    preferred_element_type=jnp.float32)
            gates = gates + bl_ref[...]
            i_g = gates[:, 0:128]
            g_g = gates[:, 256:384]
            o_g = gates[:, 384:512]
            c = jax.nn.sigmoid(i_g) * jnp.tanh(g_g)
            hh = jax.nn.sigmoid(o_g) * jnp.tanh(c)
            out_ref[...] = lax.dot_general(hh, wfc_ref[...], dt,
                                           preferred_element_type=jnp.float32
                                           ) + bfc_ref[...]

    def accsum(acc_ref, out_ref):
        a = acc_ref[...]
        out_ref[...] = (a[0] + a[1])[None]

    acc_s = pl.pallas_call(
        accsum,
        grid=(nb,),
        in_specs=[pl.BlockSpec((NC, TCBLK, 16), lambda i: (0, i, 0))],
        out_specs=pl.BlockSpec((1, TCBLK, 16), lambda i: (0, i, 0)),
        out_shape=jax.ShapeDtypeStruct((1, NN, 16), jnp.float32),
    )(acc4)

    return pl.pallas_call(
        body,
        grid=(nb,),
        in_specs=[
            pl.BlockSpec((1, TCBLK, 16), lambda i: (0, i, 0)),
            pl.BlockSpec((NC, TCBLK, 16), lambda i: (0, i, 0)),
            pl.BlockSpec((1, 16), lambda i: (0, 0)),
            pl.BlockSpec((1, 1, TCBLK), lambda i: (i, 0, 0)),
            pl.BlockSpec(Wih_p.shape, lambda i: (0, 0)),
            pl.BlockSpec((1, 512), lambda i: (0, 0)),
            pl.BlockSpec(W_fc.shape, lambda i: (0, 0)),
            pl.BlockSpec((1, 16), lambda i: (0, 0)),
        ],
        out_specs=pl.BlockSpec((16, 16), lambda i: (0, 0)),
        out_shape=jax.ShapeDtypeStruct((16, 16), jnp.float32),
        scratch_shapes=[
            pltpu.VMEM((16, 16), jnp.float32),
            pltpu.VMEM((16, 16), jnp.float32),
        ],
    )(acc_s, den4, b4p, batch3, Wih_p, bl, W_fc, bfc)


def _mk_as2(a, hd):
    heads, dout = a.shape
    eye = jnp.eye(8, dtype=jnp.float32)[:heads]
    blockdiag = (a[:, :, None] * eye[:, None, :]).reshape(heads * dout, 8)
    if heads * dout < hd:
        blockdiag = jnp.pad(blockdiag, ((0, hd - heads * dout), (0, 0)))
    return jnp.concatenate([blockdiag, blockdiag], axis=1)


def _mk_exp8(heads, dout, hd):
    col_head = jnp.minimum(jnp.arange(hd) // dout, heads - 1)
    return (jnp.arange(8)[:, None] == col_head[None, :]).astype(jnp.float32)


def kernel(x, edge_index, batch, W1, as1, ad1, b1, W2, as2, ad2, b2, W3, as3,
           ad3, b3, W4, as4, ad4, b4, W_ih, W_hh, b_ih, b_hh, W_fc, b_fc):
    f32 = jnp.float32
    src = edge_index[0].astype(jnp.int32)
    dst = edge_index[1].astype(jnp.int32)
    srcp = jnp.concatenate([src, jnp.zeros((EP - EE,), jnp.int32)])
    dstp = jnp.concatenate([dst, jnp.full((EP - EE,), NP - 1, jnp.int32)])
    sd = jnp.stack([srcp.reshape(EPB, BLK), dstp.reshape(EPB, BLK)], axis=1)
    batch3 = batch.astype(jnp.int32).reshape(NN // TCBLK, 1, TCBLK)

    W4p = jnp.pad(W4, ((0, 0), (0, 8)))
    b4p = jnp.pad(b4, (0, 8)).reshape(1, 16).astype(f32)
    Wih_p = jnp.pad(W_ih, ((0, 0), (0, 8))).astype(f32)
    bl = (b_ih + b_hh).reshape(1, 512).astype(f32)
    bfc = b_fc.reshape(1, 16).astype(f32)

    att = _make_att_kernel()
    msg1 = _make_msg_kernel(4, 128, 2, 2, False)
    msg2 = _make_msg_kernel(2, 128, 1, 4, False)
    msg3 = _make_msg_kernel(1, 128, 1, 8, True)
    msg4 = _make_msg_kernel(1, 16, 1, 1, True)

    h1, als1, ald1 = _tc_first(x, W1, _mk_as2(as1, 512), _mk_as2(ad1, 512))
    w1, den1 = att(als1, ald1, sd)
    acc1 = msg1(h1.reshape(NN * 4, 128), sd, w1)

    h2, als2, ald2 = _tc_mid(acc1, den1, _mk_exp8(8, 64, 512),
                             b1.reshape(1, 512), W2,
                             _mk_as2(as2, 256), _mk_as2(ad2, 256), False)
    w2, den2 = att(als2, ald2, sd)
    acc2 = msg2(h2.reshape(NN * 2, 128), sd, w2)

    h3, als3, ald3 = _tc_mid(acc2, den2, _mk_exp8(8, 32, 256),
                             b2.reshape(1, 256), W3,
                             _mk_as2(as3, 128), _mk_as2(ad3, 128), False)
    w3, den3 = att(als3, ald3, sd)
    acc3 = msg3(h3, sd, w3)

    h4, als4, ald4 = _tc_mid(acc3, den3, _mk_exp8(8, 16, 128),
                             b3.reshape(1, 128), W4p,
                             _mk_as2(as4, 16), _mk_as2(ad4, 16), True)
    w4, den4 = att(als4, ald4, sd)
    acc4 = msg4(h4, sd, w4)

    return _tc_final(acc4, den4, b4p, batch3, Wih_p, bl, W_fc, bfc)


# speedup vs baseline: 1.3230x; 1.0019x over previous
import functools

import jax
import jax.numpy as jnp
from jax import lax
from jax.experimental import pallas as pl
from jax.experimental.pallas import tpu as pltpu
from jax.experimental.pallas import tpu_sc as plsc

NN = 10000
NP = 10240
EE = 320000
NC = 2
NS = 16
LANES = 16
BLK = 128
EPB = 2560
EP = EPB * BLK
ROWS_PER_TEC = NP // NS
ZROWS = 128
TCBLK = 1000

_SC_PARAMS = pltpu.CompilerParams(use_tc_tiling_on_sc=False,
                                  needs_layout_passes=False)


def _make_att_kernel():
    nb = EPB // (NC * NS)
    mesh = plsc.VectorSubcoreMesh(core_axis_name="c", subcore_axis_name="s")

    @functools.partial(
        pl.kernel,
        out_type=(
            jax.ShapeDtypeStruct((EP, 16), jnp.float32),
            jax.ShapeDtypeStruct((NC, NP, 16), jnp.float32),
        ),
        mesh=mesh,
        scratch_types=(
            pltpu.VMEM((2, BLK), jnp.int32),
            pltpu.VMEM((2, BLK), jnp.int32),
            pltpu.VMEM((BLK, 16), jnp.float32),
            pltpu.VMEM((BLK, 16), jnp.float32),
            pltpu.VMEM((BLK, 16), jnp.float32),
            pltpu.VMEM((BLK, 16), jnp.float32),
            pltpu.VMEM((BLK, 16), jnp.float32),
            pltpu.VMEM((BLK, 16), jnp.float32),
            pltpu.VMEM_SHARED((NP, 16), jnp.float32),
            pltpu.SemaphoreType.DMA,
            pltpu.SemaphoreType.DMA,
            pltpu.SemaphoreType.DMA,
            pltpu.SemaphoreType.DMA,
        ),
        compiler_params=_SC_PARAMS,
    )
    def att(als_hbm, ald_hbm, sd_hbm, w_hbm, den_hbm,
            sd0, sd1, as0, as1, ad0, ad1, wr0, wr1, den_sh,
            ss0, ss1, sa0, sa1):
        core = lax.axis_index("c")
        sub = lax.axis_index("s")

        def zinit(i, _):
            as0[i, :] = jnp.zeros((LANES,), jnp.float32)
            return 0
        lax.fori_loop(0, ZROWS, zinit, 0)
        row0 = sub * ROWS_PER_TEC
        for k in range(ROWS_PER_TEC // ZROWS):
            pltpu.sync_copy(as0, den_sh.at[pl.ds(row0 + k * ZROWS, ZROWS)])
        plsc.subcore_barrier()

        blo = (core * NS + sub) * nb
        sets = ((sd0, as0, ad0, wr0, ss0, sa0),
                (sd1, as1, ad1, wr1, ss1, sa1))

        def issue(b, st):
            sd, asv, adv, _, s_s, s_a = st
            pltpu.sync_copy(sd_hbm.at[b], sd)
            pltpu.async_copy(als_hbm.at[sd.at[0]], asv, s_s)
            pltpu.async_copy(ald_hbm.at[sd.at[1]], adv, s_a)

        def process(b, st):
            sd, asv, adv, wr, s_s, s_a = st
            pltpu.make_async_copy(als_hbm.at[sd.at[0]], asv, s_s).wait()
            pltpu.make_async_copy(ald_hbm.at[sd.at[1]], adv, s_a).wait()

            def edge(i, _):
                s = asv[i, :] + adv[i, :]
                wr[i, :] = jnp.exp(jnp.maximum(s, 0.2 * s))
                return 0
            lax.fori_loop(0, BLK, edge, 0, unroll=4)
            pltpu.sync_copy(wr, w_hbm.at[pl.ds(b * BLK, BLK)])
            pltpu.sync_copy(wr, den_sh.at[sd.at[1]], add=True)

        issue(blo, sets[0])
        issue(blo + 1, sets[1])

        def pair(i, _):
            b0 = blo + 2 * i
            process(b0, sets[0])

            @pl.when(i < nb // 2 - 1)
            def _():
                issue(b0 + 2, sets[0])
            process(b0 + 1, sets[1])

            @pl.when(i < nb // 2 - 1)
            def _():
                issue(b0 + 3, sets[1])
            return 0
        lax.fori_loop(0, nb // 2, pair, 0)
        plsc.subcore_barrier()
        for k in range(ROWS_PER_TEC // ZROWS):
            r = row0 + k * ZROWS
            pltpu.sync_copy(den_sh.at[pl.ds(r, ZROWS)],
                            den_hbm.at[core, pl.ds(r, ZROWS)])

    return att


def _make_msg_kernel(nslab, cols, slabs_per_core, heads_per_slab, edge_split):
    vpr = cols // LANES
    vph = vpr // heads_per_slab
    n_out = NC if edge_split else nslab
    nb = EPB // (NC * NS) if edge_split else EPB // NS
    mesh = plsc.VectorSubcoreMesh(core_axis_name="c", subcore_axis_name="s")

    @functools.partial(
        pl.kernel,
        out_type=jax.ShapeDtypeStruct((n_out, NP, cols), jnp.float32),
        mesh=mesh,
        scratch_types=(
            pltpu.VMEM((2, BLK), jnp.int32),
            pltpu.VMEM((2, BLK), jnp.int32),
            pltpu.VMEM((BLK,), jnp.int32),
            pltpu.VMEM((BLK,), jnp.int32),
            pltpu.VMEM((BLK, 16), jnp.float32),
            pltpu.VMEM((BLK, 16), jnp.float32),
            pltpu.VMEM((BLK, cols), jnp.float32),
            pltpu.VMEM((BLK, cols), jnp.float32),
            pltpu.VMEM_SHARED((NP, cols), jnp.float32),
            pltpu.SemaphoreType.DMA,
            pltpu.SemaphoreType.DMA,
            pltpu.SemaphoreType.DMA,
            pltpu.SemaphoreType.DMA,
        ),
        compiler_params=_SC_PARAMS,
    )
    def msg(h_hbm, sd_hbm, w_hbm, out_hbm,
            sd0, sd1, gx0, gx1, wr0, wr1, rows0, rows1, acc_sh,
            sr0, sr1, sw0, sw1):
        core = lax.axis_index("c")
        sub = lax.axis_index("s")
        row0 = sub * ROWS_PER_TEC

        sets = ((sd0, gx0, wr0, rows0, sr0, sw0),
                (sd1, gx1, wr1, rows1, sr1, sw1))

        for si in range(slabs_per_core):
            slab = 0 if edge_split else core * slabs_per_core + si

            def zinit(i, _):
                for j in range(vpr):
                    rows0[i, pl.ds(j * LANES, LANES)] = jnp.zeros(
                        (LANES,), jnp.float32)
                return 0
            lax.fori_loop(0, ZROWS, zinit, 0)
            for k in range(ROWS_PER_TEC // ZROWS):
                pltpu.sync_copy(rows0,
                                acc_sh.at[pl.ds(row0 + k * ZROWS, ZROWS)])
            plsc.subcore_barrier()

            blo = ((core * NS + sub) * nb if edge_split else sub * nb)
            h0 = slab * heads_per_slab

            def gidx_ref(st):
                sd, gx = st[0], st[1]
                if nslab == 1:
                    return sd.at[0]
                return gx

            def issue(b, st):
                sd, gx, wr, rows, s_r, s_w = st
                pltpu.sync_copy(sd_hbm.at[b], sd)
                if nslab != 1:
                    for k in range(BLK // LANES):
                        sl = pl.ds(k * LANES, LANES)
                        gx[sl] = sd[0, sl] * nslab + slab
                pltpu.async_copy(h_hbm.at[gidx_ref(st)], rows, s_r)
                pltpu.async_copy(w_hbm.at[pl.ds(b * BLK, BLK)], wr, s_w)

            def process(b, st):
                sd, gx, wr, rows, s_r, s_w = st
                pltpu.make_async_copy(h_hbm.at[gidx_ref(st)], rows,
                                      s_r).wait()
                pltpu.make_async_copy(w_hbm.at[pl.ds(b * BLK, BLK)], wr,
                                      s_w).wait()

                def edge(i, _):
                    for hh in range(heads_per_slab):
                        wb = plsc.load_gather(
                            wr,
                            [jnp.full((LANES,), i, jnp.int32),
                             jnp.full((LANES,), h0 + hh, jnp.int32)])
                        for j in range(vph):
                            sl = pl.ds((hh * vph + j) * LANES, LANES)
                            rows[i, sl] = rows[i, sl] * wb
                    return 0
                lax.fori_loop(0, BLK, edge, 0, unroll=4)
                pltpu.sync_copy(rows, acc_sh.at[sd.at[1]], add=True)

            issue(blo, sets[0])
            issue(blo + 1, sets[1])

            def pair(i, _):
                b0 = blo + 2 * i
                process(b0, sets[0])

                @pl.when(i < nb // 2 - 1)
                def _():
                    issue(b0 + 2, sets[0])
                process(b0 + 1, sets[1])

                @pl.when(i < nb // 2 - 1)
                def _():
                    issue(b0 + 3, sets[1])
                return 0
            lax.fori_loop(0, nb // 2, pair, 0)
            plsc.subcore_barrier()

            out_maj = core if edge_split else slab
            for k in range(ROWS_PER_TEC // ZROWS):
                r = row0 + k * ZROWS
                pltpu.sync_copy(acc_sh.at[pl.ds(r, ZROWS)],
                                out_hbm.at[out_maj, pl.ds(r, ZROWS)])

    return msg


def _tc_first(x, W, As2, Ad2):
    hd = W.shape[1]

    def body(x_ref, w_ref, as_ref, ad_ref, h_ref, als_ref, ald_ref):
        h = jnp.dot(x_ref[...], w_ref[...], preferred_element_type=jnp.float32)
        h_ref[...] = h
        als_ref[...] = jnp.dot(h, as_ref[...],
                               preferred_element_type=jnp.float32)
        ald_ref[...] = jnp.dot(h, ad_ref[...],
                               preferred_element_type=jnp.float32)

    return pl.pallas_call(
        body,
        grid=(NN // TCBLK,),
        in_specs=[
            pl.BlockSpec((TCBLK, x.shape[1]), lambda i: (i, 0)),
            pl.BlockSpec(W.shape, lambda i: (0, 0)),
            pl.BlockSpec(As2.shape, lambda i: (0, 0)),
            pl.BlockSpec(Ad2.shape, lambda i: (0, 0)),
        ],
        out_specs=[
            pl.BlockSpec((TCBLK, hd), lambda i: (i, 0)),
            pl.BlockSpec((TCBLK, 16), lambda i: (i, 0)),
            pl.BlockSpec((TCBLK, 16), lambda i: (i, 0)),
        ],
        out_shape=[
            jax.ShapeDtypeStruct((NN, hd), jnp.float32),
            jax.ShapeDtypeStruct((NN, 16), jnp.float32),
            jax.ShapeDtypeStruct((NN, 16), jnp.float32),
        ],
    )(x, W, As2, Ad2)


def _tc_mid(acc, den, exp8, bvec, W, As2, Ad2, sum_parts):
    P = acc.shape[0]
    C = acc.shape[2]
    hd_prev = exp8.shape[1]
    hd = W.shape[1]

    def body(acc_ref, den_ref, e8_ref, b_ref, w_ref, as_ref, ad_ref,
             h_ref, als_ref, ald_ref):
        a = acc_ref[...]
        if sum_parts:
            g = a[0] + a[1]
        else:
            g = jnp.concatenate([a[p] for p in range(P)], axis=-1)
        d = den_ref[...]
        den8 = d[0, :, 0:8] + d[1, :, 0:8]
        denr = jnp.dot(den8, e8_ref[...], preferred_element_type=jnp.float32)
        g = g / (denr + 1e-16) + b_ref[...]
        g = jnp.where(g > 0, g, jnp.exp(g) - 1.0)
        h = jnp.dot(g, w_ref[...], preferred_element_type=jnp.float32)
        h_ref[...] = h
        als_ref[...] = jnp.dot(h, as_ref[...],
                               preferred_element_type=jnp.float32)
        ald_ref[...] = jnp.dot(h, ad_ref[...],
                               preferred_element_type=jnp.float32)

    return pl.pallas_call(
        body,
        grid=(NN // TCBLK,),
        in_specs=[
            pl.BlockSpec((P, TCBLK, C), lambda i: (0, i, 0)),
            pl.BlockSpec((NC, TCBLK, 16), lambda i: (0, i, 0)),
            pl.BlockSpec(exp8.shape, lambda i: (0, 0)),
            pl.BlockSpec((1, hd_prev), lambda i: (0, 0)),
            pl.BlockSpec(W.shape, lambda i: (0, 0)),
            pl.BlockSpec(As2.shape, lambda i: (0, 0)),
            pl.BlockSpec(Ad2.shape, lambda i: (0, 0)),
        ],
        out_specs=[
            pl.BlockSpec((TCBLK, hd), lambda i: (i, 0)),
            pl.BlockSpec((TCBLK, 16), lambda i: (i, 0)),
            pl.BlockSpec((TCBLK, 16), lambda i: (i, 0)),
        ],
        out_shape=[
            jax.ShapeDtypeStruct((NN, hd), jnp.float32),
            jax.ShapeDtypeStruct((NN, 16), jnp.float32),
            jax.ShapeDtypeStruct((NN, 16), jnp.float32),
        ],
    )(acc, den, exp8, bvec, W, As2, Ad2)


def _tc_final(acc4, den4, b4p, batch3, Wih_p, bl, W_fc, bfc):
    nb = NN // TCBLK

    def body(acc_ref, den_ref, b4_ref, bat_ref, wih_ref, bl_ref, wfc_ref,
             bfc_ref, out_ref, sums_ref, cnts_ref):
        i = pl.program_id(0)

        @pl.when(i == 0)
        def _():
            sums_ref[...] = jnp.zeros_like(sums_ref)
            cnts_ref[...] = jnp.zeros_like(cnts_ref)

        d = den_ref[...]
        den = d[0, :, 0:1] + d[1, :, 0:1]
        g = acc_ref[...][0] / (den + 1e-16) + b4_ref[...]
        g = jnp.where(g > 0, g, jnp.exp(g) - 1.0)
        b = bat_ref[0, 0, :]
        oh = (b[:, None] == lax.broadcasted_iota(jnp.int32, (TCBLK, 16), 1)
              ).astype(jnp.float32)
        dn = (((0,), (0,)), ((), ()))
        sums_ref[...] += lax.dot_general(oh, g, dn,
                                         preferred_element_type=jnp.float32)
        cnts_ref[...] += lax.dot_general(oh, jnp.ones_like(g), dn,
                                         preferred_element_type=jnp.float32)

        @pl.when(i == nb - 1)
        def _():
            emb = sums_ref[...] / jnp.maximum(cnts_ref[...], 1.0)
            dt = (((1,), (1,)), ((), ()))
            gates = lax.dot_general(emb, wih_ref[...], dt,
                                    preferred_element_type=jnp.float32)
            gates = gates + bl_ref[...]
            i_g = gates[:, 0:128]
            g_g = gates[:, 256:384]
            o_g = gates[:, 384:512]
            c = jax.nn.sigmoid(i_g) * jnp.tanh(g_g)
            hh = jax.nn.sigmoid(o_g) * jnp.tanh(c)
            out_ref[...] = lax.dot_general(hh, wfc_ref[...], dt,
                                           preferred_element_type=jnp.float32
                                           ) + bfc_ref[...]

    def accsum(acc_ref, out_ref):
        a = acc_ref[...]
        out_ref[...] = (a[0] + a[1])[None]

    acc_s = pl.pallas_call(
        accsum,
        grid=(nb,),
        in_specs=[pl.BlockSpec((NC, TCBLK, 16), lambda i: (0, i, 0))],
        out_specs=pl.BlockSpec((1, TCBLK, 16), lambda i: (0, i, 0)),
        out_shape=jax.ShapeDtypeStruct((1, NN, 16), jnp.float32),
    )(acc4)

    return pl.pallas_call(
        body,
        grid=(nb,),
        in_specs=[
            pl.BlockSpec((1, TCBLK, 16), lambda i: (0, i, 0)),
            pl.BlockSpec((NC, TCBLK, 16), lambda i: (0, i, 0)),
            pl.BlockSpec((1, 16), lambda i: (0, 0)),
            pl.BlockSpec((1, 1, TCBLK), lambda i: (i, 0, 0)),
            pl.BlockSpec(Wih_p.shape, lambda i: (0, 0)),
            pl.BlockSpec((1, 512), lambda i: (0, 0)),
            pl.BlockSpec(W_fc.shape, lambda i: (0, 0)),
            pl.BlockSpec((1, 16), lambda i: (0, 0)),
        ],
        out_specs=pl.BlockSpec((16, 16), lambda i: (0, 0)),
        out_shape=jax.ShapeDtypeStruct((16, 16), jnp.float32),
        scratch_shapes=[
            pltpu.VMEM((16, 16), jnp.float32),
            pltpu.VMEM((16, 16), jnp.float32),
        ],
    )(acc_s, den4, b4p, batch3, Wih_p, bl, W_fc, bfc)


def _mk_as2(a, hd):
    heads, dout = a.shape
    eye = jnp.eye(8, dtype=jnp.float32)[:heads]
    blockdiag = (a[:, :, None] * eye[:, None, :]).reshape(heads * dout, 8)
    if heads * dout < hd:
        blockdiag = jnp.pad(blockdiag, ((0, hd - heads * dout), (0, 0)))
    return jnp.concatenate([blockdiag, blockdiag], axis=1)


def _mk_exp8(heads, dout, hd):
    col_head = jnp.minimum(jnp.arange(hd) // dout, heads - 1)
    return (jnp.arange(8)[:, None] == col_head[None, :]).astype(jnp.float32)


def kernel(x, edge_index, batch, W1, as1, ad1, b1, W2, as2, ad2, b2, W3, as3,
           ad3, b3, W4, as4, ad4, b4, W_ih, W_hh, b_ih, b_hh, W_fc, b_fc):
    f32 = jnp.float32
    src = edge_index[0].astype(jnp.int32)
    dst = edge_index[1].astype(jnp.int32)
    srcp = jnp.concatenate([src, jnp.zeros((EP - EE,), jnp.int32)])
    dstp = jnp.concatenate([dst, jnp.full((EP - EE,), NP - 1, jnp.int32)])
    sd = jnp.stack([srcp.reshape(EPB, BLK), dstp.reshape(EPB, BLK)], axis=1)
    batch3 = batch.astype(jnp.int32).reshape(NN // TCBLK, 1, TCBLK)

    W4p = jnp.pad(W4, ((0, 0), (0, 8)))
    b4p = jnp.pad(b4, (0, 8)).reshape(1, 16).astype(f32)
    Wih_p = jnp.pad(W_ih, ((0, 0), (0, 8))).astype(f32)
    bl = (b_ih + b_hh).reshape(1, 512).astype(f32)
    bfc = b_fc.reshape(1, 16).astype(f32)

    att = _make_att_kernel()
    msg1 = _make_msg_kernel(4, 128, 2, 2, False)
    msg2 = _make_msg_kernel(2, 128, 1, 4, False)
    msg3 = _make_msg_kernel(1, 128, 1, 8, True)
    msg4 = _make_msg_kernel(1, 16, 1, 1, True)

    h1, als1, ald1 = _tc_first(x, W1, _mk_as2(as1, 512), _mk_as2(ad1, 512))
    w1, den1 = att(als1, ald1, sd)
    acc1 = msg1(h1.reshape(NN * 4, 128), sd, w1)

    h2, als2, ald2 = _tc_mid(acc1, den1, _mk_exp8(8, 64, 512),
                             b1.reshape(1, 512), W2,
                             _mk_as2(as2, 256), _mk_as2(ad2, 256), False)
    w2, den2 = att(als2, ald2, sd)
    acc2 = msg2(h2.reshape(NN * 2, 128), sd, w2)

    h3, als3, ald3 = _tc_mid(acc2, den2, _mk_exp8(8, 32, 256),
                             b2.reshape(1, 256), W3,
                             _mk_as2(as3, 128), _mk_as2(ad3, 128), False)
    w3, den3 = att(als3, ald3, sd)
    acc3 = msg3(h3, sd, w3)

    h4, als4, ald4 = _tc_mid(acc3, den3, _mk_exp8(8, 16, 128),
                             b3.reshape(1, 128), W4p,
                             _mk_as2(as4, 16), _mk_as2(ad4, 16), True)
    w4, den4 = att(als4, ald4, sd)
    acc4 = msg4(h4, sd, w4)

    return _tc_final(acc4, den4, b4p, batch3, Wih_p, bl, W_fc, bfc)


# speedup vs baseline: 1.4480x; 1.0945x over previous
import functools

import jax
import jax.numpy as jnp
from jax import lax
from jax.experimental import pallas as pl
from jax.experimental.pallas import tpu as pltpu
from jax.experimental.pallas import tpu_sc as plsc

NN = 10000
NP = 10240
EE = 320000
NC = 2
NS = 16
LANES = 16
BLK = 128
EPB = 2560
EP = EPB * BLK
ROWS_PER_TEC = NP // NS
ZROWS = 128
TCBLK = 1000

_SC_PARAMS = pltpu.CompilerParams(use_tc_tiling_on_sc=False,
                                  needs_layout_passes=False)


def _make_att_kernel():
    nb = EPB // (NC * NS)
    mesh = plsc.VectorSubcoreMesh(core_axis_name="c", subcore_axis_name="s")

    @functools.partial(
        pl.kernel,
        out_type=(
            jax.ShapeDtypeStruct((EP, 16), jnp.float32),
            jax.ShapeDtypeStruct((NC, NP, 16), jnp.float32),
        ),
        mesh=mesh,
        scratch_types=(
            pltpu.VMEM((2, BLK), jnp.int32),
            pltpu.VMEM((2, BLK), jnp.int32),
            pltpu.VMEM((BLK, 16), jnp.float32),
            pltpu.VMEM((BLK, 16), jnp.float32),
            pltpu.VMEM((BLK, 16), jnp.float32),
            pltpu.VMEM((BLK, 16), jnp.float32),
            pltpu.VMEM((BLK, 16), jnp.float32),
            pltpu.VMEM((BLK, 16), jnp.float32),
            pltpu.VMEM_SHARED((NP, 16), jnp.float32),
            pltpu.SemaphoreType.DMA,
            pltpu.SemaphoreType.DMA,
            pltpu.SemaphoreType.DMA,
            pltpu.SemaphoreType.DMA,
        ),
        compiler_params=_SC_PARAMS,
    )
    def att(als_hbm, ald_hbm, sd_hbm, w_hbm, den_hbm,
            sd0, sd1, as0, as1, ad0, ad1, wr0, wr1, den_sh,
            ss0, ss1, sa0, sa1):
        core = lax.axis_index("c")
        sub = lax.axis_index("s")

        def zinit(i, _):
            as0[i, :] = jnp.zeros((LANES,), jnp.float32)
            return 0
        lax.fori_loop(0, ZROWS, zinit, 0)
        row0 = sub * ROWS_PER_TEC
        for k in range(ROWS_PER_TEC // ZROWS):
            pltpu.sync_copy(as0, den_sh.at[pl.ds(row0 + k * ZROWS, ZROWS)])
        plsc.subcore_barrier()

        blo = (core * NS + sub) * nb
        sets = ((sd0, as0, ad0, wr0, ss0, sa0),
                (sd1, as1, ad1, wr1, ss1, sa1))

        def issue(b, st):
            sd, asv, adv, _, s_s, s_a = st
            pltpu.sync_copy(sd_hbm.at[b], sd)
            pltpu.async_copy(als_hbm.at[sd.at[0]], asv, s_s)
            pltpu.async_copy(ald_hbm.at[sd.at[1]], adv, s_a)

        def process(b, st):
            sd, asv, adv, wr, s_s, s_a = st
            pltpu.make_async_copy(als_hbm.at[sd.at[0]], asv, s_s).wait()
            pltpu.make_async_copy(ald_hbm.at[sd.at[1]], adv, s_a).wait()

            def edge(i, _):
                s = asv[i, :] + adv[i, :]
                wr[i, :] = jnp.exp(jnp.maximum(s, 0.2 * s))
                return 0
            lax.fori_loop(0, BLK, edge, 0, unroll=4)
            pltpu.sync_copy(wr, w_hbm.at[pl.ds(b * BLK, BLK)])
            pltpu.sync_copy(wr, den_sh.at[sd.at[1]], add=True)

        issue(blo, sets[0])
        issue(blo + 1, sets[1])

        def pair(i, _):
            b0 = blo + 2 * i
            process(b0, sets[0])

            @pl.when(i < nb // 2 - 1)
            def _():
                issue(b0 + 2, sets[0])
            process(b0 + 1, sets[1])

            @pl.when(i < nb // 2 - 1)
            def _():
                issue(b0 + 3, sets[1])
            return 0
        lax.fori_loop(0, nb // 2, pair, 0)
        plsc.subcore_barrier()
        for k in range(ROWS_PER_TEC // ZROWS):
            r = row0 + k * ZROWS
            pltpu.sync_copy(den_sh.at[pl.ds(r, ZROWS)],
                            den_hbm.at[core, pl.ds(r, ZROWS)])

    return att


def _make_msg_kernel(nslab, cols, slabs_per_core, heads_per_slab, edge_split,
                     packed=True):
    vpr = cols // LANES
    vph = vpr // heads_per_slab
    gw = cols // 2 if packed else cols
    gdt = jnp.int32 if packed else jnp.float32
    ch = cols // heads_per_slab
    n_out = NC if edge_split else nslab
    nb = EPB // (NC * NS) if edge_split else EPB // NS
    mesh = plsc.VectorSubcoreMesh(core_axis_name="c", subcore_axis_name="s")

    @functools.partial(
        pl.kernel,
        out_type=jax.ShapeDtypeStruct((n_out, NP, cols), jnp.float32),
        mesh=mesh,
        scratch_types=(
            pltpu.VMEM((2, BLK), jnp.int32),
            pltpu.VMEM((2, BLK), jnp.int32),
            pltpu.VMEM((BLK,), jnp.int32),
            pltpu.VMEM((BLK,), jnp.int32),
            pltpu.VMEM((BLK, 16), jnp.float32),
            pltpu.VMEM((BLK, 16), jnp.float32),
            pltpu.VMEM((BLK, gw), gdt),
            pltpu.VMEM((BLK, gw), gdt),
            pltpu.VMEM((BLK, cols), jnp.float32),
            pltpu.VMEM_SHARED((NP, cols), jnp.float32),
            pltpu.SemaphoreType.DMA,
            pltpu.SemaphoreType.DMA,
            pltpu.SemaphoreType.DMA,
            pltpu.SemaphoreType.DMA,
        ),
        compiler_params=_SC_PARAMS,
    )
    def msg(h_hbm, sd_hbm, w_hbm, out_hbm,
            sd0, sd1, gx0, gx1, wr0, wr1, rows0, rows1, rowsf, acc_sh,
            sr0, sr1, sw0, sw1):
        core = lax.axis_index("c")
        sub = lax.axis_index("s")
        row0 = sub * ROWS_PER_TEC

        sets = ((sd0, gx0, wr0, rows0, sr0, sw0),
                (sd1, gx1, wr1, rows1, sr1, sw1))

        for si in range(slabs_per_core):
            slab = 0 if edge_split else core * slabs_per_core + si

            def zinit(i, _):
                for j in range(vpr):
                    rowsf[i, pl.ds(j * LANES, LANES)] = jnp.zeros(
                        (LANES,), jnp.float32)
                return 0
            lax.fori_loop(0, ZROWS, zinit, 0)
            for k in range(ROWS_PER_TEC // ZROWS):
                pltpu.sync_copy(rowsf,
                                acc_sh.at[pl.ds(row0 + k * ZROWS, ZROWS)])
            plsc.subcore_barrier()

            blo = ((core * NS + sub) * nb if edge_split else sub * nb)
            h0 = slab * heads_per_slab

            def gidx_ref(st):
                sd, gx = st[0], st[1]
                if nslab == 1:
                    return sd.at[0]
                return gx

            def issue(b, st):
                sd, gx, wr, rows, s_r, s_w = st
                pltpu.sync_copy(sd_hbm.at[b], sd)
                if nslab != 1:
                    for k in range(BLK // LANES):
                        sl = pl.ds(k * LANES, LANES)
                        gx[sl] = sd[0, sl] * nslab + slab
                pltpu.async_copy(h_hbm.at[gidx_ref(st)], rows, s_r)
                pltpu.async_copy(w_hbm.at[pl.ds(b * BLK, BLK)], wr, s_w)

            def process(b, st):
                sd, gx, wr, rows, s_r, s_w = st
                pltpu.make_async_copy(h_hbm.at[gidx_ref(st)], rows,
                                      s_r).wait()
                pltpu.make_async_copy(w_hbm.at[pl.ds(b * BLK, BLK)], wr,
                                      s_w).wait()

                if packed:
                    def edge(i, _):
                        wbs = [plsc.load_gather(
                            wr,
                            [jnp.full((LANES,), i, jnp.int32),
                             jnp.full((LANES,), h0 + hh, jnp.int32)])
                            for hh in range(heads_per_slab)]
                        for g in range(cols // 32):
                            p = rows[i, pl.ds(g * LANES, LANES)]
                            af = lax.bitcast_convert_type(
                                p & jnp.int32(-65536), jnp.float32)
                            bf = lax.bitcast_convert_type(
                                p << 16, jnp.float32)
                            rowsf[i, pl.ds(g * 32, LANES)] = (
                                af * wbs[(g * 32) // ch])
                            rowsf[i, pl.ds(g * 32 + 16, LANES)] = (
                                bf * wbs[(g * 32 + 16) // ch])
                        return 0
                    lax.fori_loop(0, BLK, edge, 0, unroll=4)
                    pltpu.sync_copy(rowsf, acc_sh.at[sd.at[1]], add=True)
                else:
                    def edge(i, _):
                        for hh in range(heads_per_slab):
                            wb = plsc.load_gather(
                                wr,
                                [jnp.full((LANES,), i, jnp.int32),
                                 jnp.full((LANES,), h0 + hh, jnp.int32)])
                            for j in range(vph):
                                sl = pl.ds((hh * vph + j) * LANES, LANES)
                                rows[i, sl] = rows[i, sl] * wb
                        return 0
                    lax.fori_loop(0, BLK, edge, 0, unroll=4)
                    pltpu.sync_copy(rows, acc_sh.at[sd.at[1]], add=True)

            issue(blo, sets[0])
            issue(blo + 1, sets[1])

            def pair(i, _):
                b0 = blo + 2 * i
                process(b0, sets[0])

                @pl.when(i < nb // 2 - 1)
                def _():
                    issue(b0 + 2, sets[0])
                process(b0 + 1, sets[1])

                @pl.when(i < nb // 2 - 1)
                def _():
                    issue(b0 + 3, sets[1])
                return 0
            lax.fori_loop(0, nb // 2, pair, 0)
            plsc.subcore_barrier()

            out_maj = core if edge_split else slab
            for k in range(ROWS_PER_TEC // ZROWS):
                r = row0 + k * ZROWS
                pltpu.sync_copy(acc_sh.at[pl.ds(r, ZROWS)],
                                out_hbm.at[out_maj, pl.ds(r, ZROWS)])

    return msg


def _pack_h(h, hd):
    bits = lax.bitcast_convert_type(h, jnp.int32)
    outs = []
    for g in range(hd // 32):
        a = bits[:, g * 32:g * 32 + 16]
        b = bits[:, g * 32 + 16:g * 32 + 32]
        ar = (a + 0x8000) & jnp.int32(-65536)
        br = ((b + 0x8000) >> 16) & 0xFFFF
        outs.append(ar | br)
    return jnp.concatenate(outs, axis=1)


def _tc_first(x, W, As2, Ad2):
    hd = W.shape[1]

    def body(x_ref, w_ref, as_ref, ad_ref, hp_ref, als_ref, ald_ref):
        h = jnp.dot(x_ref[...], w_ref[...], preferred_element_type=jnp.float32)
        hp_ref[...] = _pack_h(h, hd)
        als_ref[...] = jnp.dot(h, as_ref[...],
                               preferred_element_type=jnp.float32)
        ald_ref[...] = jnp.dot(h, ad_ref[...],
                               preferred_element_type=jnp.float32)

    return pl.pallas_call(
        body,
        grid=(NN // TCBLK,),
        in_specs=[
            pl.BlockSpec((TCBLK, x.shape[1]), lambda i: (i, 0)),
            pl.BlockSpec(W.shape, lambda i: (0, 0)),
            pl.BlockSpec(As2.shape, lambda i: (0, 0)),
            pl.BlockSpec(Ad2.shape, lambda i: (0, 0)),
        ],
        out_specs=[
            pl.BlockSpec((TCBLK, hd // 2), lambda i: (i, 0)),
            pl.BlockSpec((TCBLK, 16), lambda i: (i, 0)),
            pl.BlockSpec((TCBLK, 16), lambda i: (i, 0)),
        ],
        out_shape=[
            jax.ShapeDtypeStruct((NN, hd // 2), jnp.int32),
            jax.ShapeDtypeStruct((NN, 16), jnp.float32),
            jax.ShapeDtypeStruct((NN, 16), jnp.float32),
        ],
    )(x, W, As2, Ad2)


def _tc_mid(acc, den, exp8, bvec, W, As2, Ad2, sum_parts, emit_packed=True):
    P = acc.shape[0]
    C = acc.shape[2]
    hd_prev = exp8.shape[1]
    hd = W.shape[1]
    hcols = hd // 2 if emit_packed else hd
    hdt = jnp.int32 if emit_packed else jnp.float32

    def body(acc_ref, den_ref, e8_ref, b_ref, w_ref, as_ref, ad_ref,
             h_ref, als_ref, ald_ref):
        a = acc_ref[...]
        if sum_parts:
            g = a[0] + a[1]
        else:
            g = jnp.concatenate([a[p] for p in range(P)], axis=-1)
        d = den_ref[...]
        den8 = d[0, :, 0:8] + d[1, :, 0:8]
        denr = jnp.dot(den8, e8_ref[...], preferred_element_type=jnp.float32)
        g = g / (denr + 1e-16) + b_ref[...]
        g = jnp.where(g > 0, g, jnp.exp(g) - 1.0)
        h = jnp.dot(g, w_ref[...], preferred_element_type=jnp.float32)
        if emit_packed:
            h_ref[...] = _pack_h(h, hd)
        else:
            h_ref[...] = h
        als_ref[...] = jnp.dot(h, as_ref[...],
                               preferred_element_type=jnp.float32)
        ald_ref[...] = jnp.dot(h, ad_ref[...],
                               preferred_element_type=jnp.float32)

    return pl.pallas_call(
        body,
        grid=(NN // TCBLK,),
        in_specs=[
            pl.BlockSpec((P, TCBLK, C), lambda i: (0, i, 0)),
            pl.BlockSpec((NC, TCBLK, 16), lambda i: (0, i, 0)),
            pl.BlockSpec(exp8.shape, lambda i: (0, 0)),
            pl.BlockSpec((1, hd_prev), lambda i: (0, 0)),
            pl.BlockSpec(W.shape, lambda i: (0, 0)),
            pl.BlockSpec(As2.shape, lambda i: (0, 0)),
            pl.BlockSpec(Ad2.shape, lambda i: (0, 0)),
        ],
        out_specs=[
            pl.BlockSpec((TCBLK, hcols), lambda i: (i, 0)),
            pl.BlockSpec((TCBLK, 16), lambda i: (i, 0)),
            pl.BlockSpec((TCBLK, 16), lambda i: (i, 0)),
        ],
        out_shape=[
            jax.ShapeDtypeStruct((NN, hcols), hdt),
            jax.ShapeDtypeStruct((NN, 16), jnp.float32),
            jax.ShapeDtypeStruct((NN, 16), jnp.float32),
        ],
    )(acc, den, exp8, bvec, W, As2, Ad2)


def _tc_final(acc4, den4, b4p, batch3, Wih_p, bl, W_fc, bfc):
    nb = NN // TCBLK

    def body(acc_ref, den_ref, b4_ref, bat_ref, wih_ref, bl_ref, wfc_ref,
             bfc_ref, out_ref, sums_ref, cnts_ref):
        i = pl.program_id(0)

        @pl.when(i == 0)
        def _():
            sums_ref[...] = jnp.zeros_like(sums_ref)
            cnts_ref[...] = jnp.zeros_like(cnts_ref)

        d = den_ref[...]
        den = d[0, :, 0:1] + d[1, :, 0:1]
        g = acc_ref[...][0] / (den + 1e-16) + b4_ref[...]
        g = jnp.where(g > 0, g, jnp.exp(g) - 1.0)
        b = bat_ref[0, 0, :]
        oh = (b[:, None] == lax.broadcasted_iota(jnp.int32, (TCBLK, 16), 1)
              ).astype(jnp.float32)
        dn = (((0,), (0,)), ((), ()))
        sums_ref[...] += lax.dot_general(oh, g, dn,
                                         preferred_element_type=jnp.float32)
        cnts_ref[...] += lax.dot_general(oh, jnp.ones_like(g), dn,
                                         preferred_element_type=jnp.float32)

        @pl.when(i == nb - 1)
        def _():
            emb = sums_ref[...] / jnp.maximum(cnts_ref[...], 1.0)
            dt = (((1,), (1,)), ((), ()))
            gates = lax.dot_general(emb, wih_ref[...], dt,
                                    preferred_element_type=jnp.float32)
            gates = gates + bl_ref[...]
            i_g = gates[:, 0:128]
            g_g = gates[:, 256:384]
            o_g = gates[:, 384:512]
            c = jax.nn.sigmoid(i_g) * jnp.tanh(g_g)
            hh = jax.nn.sigmoid(o_g) * jnp.tanh(c)
            out_ref[...] = lax.dot_general(hh, wfc_ref[...], dt,
                                           preferred_element_type=jnp.float32
                                           ) + bfc_ref[...]

    def accsum(acc_ref, out_ref):
        a = acc_ref[...]
        out_ref[...] = (a[0] + a[1])[None]

    acc_s = pl.pallas_call(
        accsum,
        grid=(nb,),
        in_specs=[pl.BlockSpec((NC, TCBLK, 16), lambda i: (0, i, 0))],
        out_specs=pl.BlockSpec((1, TCBLK, 16), lambda i: (0, i, 0)),
        out_shape=jax.ShapeDtypeStruct((1, NN, 16), jnp.float32),
    )(acc4)

    return pl.pallas_call(
        body,
        grid=(nb,),
        in_specs=[
            pl.BlockSpec((1, TCBLK, 16), lambda i: (0, i, 0)),
            pl.BlockSpec((NC, TCBLK, 16), lambda i: (0, i, 0)),
            pl.BlockSpec((1, 16), lambda i: (0, 0)),
            pl.BlockSpec((1, 1, TCBLK), lambda i: (i, 0, 0)),
            pl.BlockSpec(Wih_p.shape, lambda i: (0, 0)),
            pl.BlockSpec((1, 512), lambda i: (0, 0)),
            pl.BlockSpec(W_fc.shape, lambda i: (0, 0)),
            pl.BlockSpec((1, 16), lambda i: (0, 0)),
        ],
        out_specs=pl.BlockSpec((16, 16), lambda i: (0, 0)),
        out_shape=jax.ShapeDtypeStruct((16, 16), jnp.float32),
        scratch_shapes=[
            pltpu.VMEM((16, 16), jnp.float32),
            pltpu.VMEM((16, 16), jnp.float32),
        ],
    )(acc_s, den4, b4p, batch3, Wih_p, bl, W_fc, bfc)


def _mk_as2(a, hd):
    heads, dout = a.shape
    eye = jnp.eye(8, dtype=jnp.float32)[:heads]
    blockdiag = (a[:, :, None] * eye[:, None, :]).reshape(heads * dout, 8)
    if heads * dout < hd:
        blockdiag = jnp.pad(blockdiag, ((0, hd - heads * dout), (0, 0)))
    return jnp.concatenate([blockdiag, blockdiag], axis=1)


def _mk_exp8(heads, dout, hd):
    col_head = jnp.minimum(jnp.arange(hd) // dout, heads - 1)
    return (jnp.arange(8)[:, None] == col_head[None, :]).astype(jnp.float32)


def kernel(x, edge_index, batch, W1, as1, ad1, b1, W2, as2, ad2, b2, W3, as3,
           ad3, b3, W4, as4, ad4, b4, W_ih, W_hh, b_ih, b_hh, W_fc, b_fc):
    f32 = jnp.float32
    src = edge_index[0].astype(jnp.int32)
    dst = edge_index[1].astype(jnp.int32)
    srcp = jnp.concatenate([src, jnp.zeros((EP - EE,), jnp.int32)])
    dstp = jnp.concatenate([dst, jnp.full((EP - EE,), NP - 1, jnp.int32)])
    sd = jnp.stack([srcp.reshape(EPB, BLK), dstp.reshape(EPB, BLK)], axis=1)
    batch3 = batch.astype(jnp.int32).reshape(NN // TCBLK, 1, TCBLK)

    W4p = jnp.pad(W4, ((0, 0), (0, 8)))
    b4p = jnp.pad(b4, (0, 8)).reshape(1, 16).astype(f32)
    Wih_p = jnp.pad(W_ih, ((0, 0), (0, 8))).astype(f32)
    bl = (b_ih + b_hh).reshape(1, 512).astype(f32)
    bfc = b_fc.reshape(1, 16).astype(f32)

    att = _make_att_kernel()
    msg1 = _make_msg_kernel(4, 128, 2, 2, False, True)
    msg2 = _make_msg_kernel(2, 128, 1, 4, False, True)
    msg3 = _make_msg_kernel(1, 128, 1, 8, True, True)
    msg4 = _make_msg_kernel(1, 16, 1, 1, True, False)

    hp1, als1, ald1 = _tc_first(x, W1, _mk_as2(as1, 512), _mk_as2(ad1, 512))
    w1, den1 = att(als1, ald1, sd)
    acc1 = msg1(hp1.reshape(NN * 4, 64), sd, w1)

    hp2, als2, ald2 = _tc_mid(acc1, den1, _mk_exp8(8, 64, 512),
                              b1.reshape(1, 512), W2,
                              _mk_as2(as2, 256), _mk_as2(ad2, 256), False)
    w2, den2 = att(als2, ald2, sd)
    acc2 = msg2(hp2.reshape(NN * 2, 64), sd, w2)

    hp3, als3, ald3 = _tc_mid(acc2, den2, _mk_exp8(8, 32, 256),
                              b2.reshape(1, 256), W3,
                              _mk_as2(as3, 128), _mk_as2(ad3, 128), False)
    w3, den3 = att(als3, ald3, sd)
    acc3 = msg3(hp3, sd, w3)

    h4, als4, ald4 = _tc_mid(acc3, den3, _mk_exp8(8, 16, 128),
                             b3.reshape(1, 128), W4p,
                             _mk_as2(as4, 16), _mk_as2(ad4, 16), True,
                             emit_packed=False)
    w4, den4 = att(als4, ald4, sd)
    acc4 = msg4(h4, sd, w4)

    return _tc_final(acc4, den4, b4p, batch3, Wih_p, bl, W_fc, bfc)
